# baseline (speedup 1.0000x reference)
import functools

import jax
import jax.numpy as jnp
from jax import lax
from jax.experimental import pallas as pl
from jax.experimental.pallas import tpu as pltpu
from jax.experimental.pallas import tpu_sc as plsc

N_NODES = 10000
N_EDGES = 320000
D = 128

NC = 2
NS = 16
NW = NC * NS
LANES = 16

CHUNK = 64
N_CHUNKS = 160
NBUF = 4
NQ = N_CHUNKS // NBUF
E_PAD = NW * N_CHUNKS * CHUNK - N_EDGES
ROWS_MAIN = 624
ROWS_TAIL = N_NODES - NS * ROWS_MAIN
D_GROUPS = D // LANES


def _matmul_call(x, w):
    def mm_body(x_ref, w_ref, o_ref):
        o_ref[...] = jnp.dot(x_ref[...], w_ref[...],
                             preferred_element_type=jnp.float32)

    return pl.pallas_call(
        mm_body,
        grid=(10,),
        in_specs=[
            pl.BlockSpec((N_NODES // 10, D), lambda i: (i, 0)),
            pl.BlockSpec((D, D), lambda i: (0, 0)),
        ],
        out_specs=pl.BlockSpec((N_NODES // 10, D), lambda i: (i, 0)),
        out_shape=jax.ShapeDtypeStruct((N_NODES, D), jnp.float32),
    )(x, w)


def _add_call(a, b):
    def add_body(a_ref, b_ref, o_ref):
        o_ref[...] = a_ref[...] + b_ref[...]

    return pl.pallas_call(
        add_body,
        grid=(10,),
        in_specs=[
            pl.BlockSpec((N_NODES // 10, D), lambda i: (i, 0)),
            pl.BlockSpec((N_NODES // 10, D), lambda i: (i, 0)),
        ],
        out_specs=pl.BlockSpec((N_NODES // 10, D), lambda i: (i, 0)),
        out_shape=jax.ShapeDtypeStruct((N_NODES, D), jnp.float32),
    )(a, b)


@functools.partial(
    pl.kernel,
    out_type=jax.ShapeDtypeStruct((NC * N_NODES, D), jnp.float32),
    mesh=plsc.VectorSubcoreMesh(core_axis_name="c", subcore_axis_name="s"),
    scratch_types=[
        [pltpu.VMEM((2, CHUNK), jnp.int32) for _ in range(NBUF)],
        [pltpu.VMEM((CHUNK,), jnp.float32) for _ in range(NBUF)],
        [pltpu.VMEM((CHUNK, D), jnp.float32) for _ in range(NBUF)],
        pltpu.VMEM_SHARED((N_NODES, D), jnp.float32),
        [pltpu.SemaphoreType.DMA for _ in range(NBUF)],
    ],
)
def _sc_spmm(ed_hbm, ev_hbm, support_hbm, out_hbm,
             ed, ev, rows, acc_sh, sem_g):
    c = lax.axis_index("c")
    s = lax.axis_index("s")
    wid = c * NS + s

    def zero_row(e, _):
        for j in range(D_GROUPS):
            rows[0][e, pl.ds(j * LANES, LANES)] = jnp.zeros((LANES,),
                                                            jnp.float32)
        return _

    lax.fori_loop(0, CHUNK, zero_row, None)
    base = s * ROWS_MAIN
    for t in range(ROWS_MAIN // CHUNK):
        pltpu.sync_copy(rows[0],
                        acc_sh.at[pl.ds(base + t * CHUNK, CHUNK)])
    rem = ROWS_MAIN % CHUNK
    pltpu.sync_copy(rows[0].at[pl.ds(0, rem)],
                    acc_sh.at[pl.ds(base + ROWS_MAIN - rem, rem)])

    @pl.when(s == 0)
    def _zero_tail():
        pltpu.sync_copy(rows[0].at[pl.ds(0, ROWS_TAIL)],
                        acc_sh.at[pl.ds(NS * ROWS_MAIN, ROWS_TAIL)])

    plsc.subcore_barrier()

    def scale(rows_b, ev_b):
        def g_body(g, _):
            evg = ev_b[pl.ds(g * LANES, LANES)]
            e0 = g * LANES
            for l in range(LANES):
                val = evg[l]
                for j in range(D_GROUPS):
                    sl = pl.ds(j * LANES, LANES)
                    rows_b[e0 + l, sl] = rows_b[e0 + l, sl] * val
            return _

        lax.fori_loop(0, CHUNK // LANES, g_body, None)

    for b in range(NBUF):
        pltpu.sync_copy(ev_hbm.at[wid, b], ev[b])
        pltpu.sync_copy(ed_hbm.at[wid, b], ed[b])
        pltpu.async_copy(support_hbm.at[ed[b].at[0]], rows[b], sem_g[b])

    def quad_body(q, _):
        i0 = NBUF * q
        for b in range(NBUF):
            pltpu.make_async_copy(support_hbm.at[ed[b].at[0]], rows[b],
                                  sem_g[b]).wait()
            scale(rows[b], ev[b])
            pltpu.sync_copy(rows[b], acc_sh.at[ed[b].at[1]], add=True)

            @pl.when(q < NQ - 1)
            def _preload():
                pltpu.sync_copy(ev_hbm.at[wid, i0 + NBUF + b], ev[b])
                pltpu.sync_copy(ed_hbm.at[wid, i0 + NBUF + b], ed[b])
                pltpu.async_copy(support_hbm.at[ed[b].at[0]], rows[b],
                                 sem_g[b])

        return _

    lax.fori_loop(0, NQ, quad_body, None)
    plsc.subcore_barrier()

    out_base = c * N_NODES + base
    for t in range(ROWS_MAIN // CHUNK):
        pltpu.sync_copy(acc_sh.at[pl.ds(base + t * CHUNK, CHUNK)],
                        rows[t % NBUF])
        pltpu.sync_copy(rows[t % NBUF],
                        out_hbm.at[pl.ds(out_base + t * CHUNK, CHUNK)])
    pltpu.sync_copy(acc_sh.at[pl.ds(base + ROWS_MAIN - rem, rem)],
                    rows[0].at[pl.ds(0, rem)])
    pltpu.sync_copy(rows[0].at[pl.ds(0, rem)],
                    out_hbm.at[pl.ds(out_base + ROWS_MAIN - rem, rem)])

    @pl.when(s == 0)
    def _drain_tail():
        pltpu.sync_copy(acc_sh.at[pl.ds(NS * ROWS_MAIN, ROWS_TAIL)],
                        rows[1].at[pl.ds(0, ROWS_TAIL)])
        pltpu.sync_copy(rows[1].at[pl.ds(0, ROWS_TAIL)],
                        out_hbm.at[pl.ds(c * N_NODES + NS * ROWS_MAIN,
                                         ROWS_TAIL)])


def kernel(edge_index, edge_values, input_feature, weight):
    zi = jnp.zeros((E_PAD,), jnp.int32)
    row = jnp.concatenate([edge_index[0].astype(jnp.int32), zi])
    col = jnp.concatenate([edge_index[1].astype(jnp.int32), zi])
    ev = jnp.concatenate([edge_values, jnp.zeros((E_PAD,), jnp.float32)])
    shp = (NW, N_CHUNKS, CHUNK)
    ed = jnp.stack([col.reshape(shp), row.reshape(shp)], axis=2)
    support = _matmul_call(input_feature, weight)
    partials = _sc_spmm(ed, ev.reshape(shp), support)
    return _add_call(partials[:N_NODES], partials[N_NODES:])


# speedup vs baseline: 2.1848x; 2.1848x over previous
import functools

import jax
import jax.numpy as jnp
from jax import lax
from jax.experimental import pallas as pl
from jax.experimental.pallas import tpu as pltpu
from jax.experimental.pallas import tpu_sc as plsc

N_NODES = 10000
N_EDGES = 320000
D = 128

NC = 2
NS = 16
NW = NC * NS
LANES = 16

CHUNK = 128
N_CHUNKS = 80
NPAIRS = N_CHUNKS // 2
E_PAD = NW * N_CHUNKS * CHUNK - N_EDGES
ROWS_MAIN = 624
ROWS_TAIL = N_NODES - NS * ROWS_MAIN
D_GROUPS = D // LANES


def _matmul_call(x, w):
    def mm_body(x_ref, w_ref, o_ref):
        o_ref[...] = jnp.dot(x_ref[...], w_ref[...],
                             preferred_element_type=jnp.float32)

    return pl.pallas_call(
        mm_body,
        grid=(10,),
        in_specs=[
            pl.BlockSpec((N_NODES // 10, D), lambda i: (i, 0)),
            pl.BlockSpec((D, D), lambda i: (0, 0)),
        ],
        out_specs=pl.BlockSpec((N_NODES // 10, D), lambda i: (i, 0)),
        out_shape=jax.ShapeDtypeStruct((N_NODES, D), jnp.float32),
    )(x, w)


def _add_call(a, b):
    def add_body(a_ref, b_ref, o_ref):
        o_ref[...] = a_ref[...] + b_ref[...]

    return pl.pallas_call(
        add_body,
        grid=(10,),
        in_specs=[
            pl.BlockSpec((N_NODES // 10, D), lambda i: (i, 0)),
            pl.BlockSpec((N_NODES // 10, D), lambda i: (i, 0)),
        ],
        out_specs=pl.BlockSpec((N_NODES // 10, D), lambda i: (i, 0)),
        out_shape=jax.ShapeDtypeStruct((N_NODES, D), jnp.float32),
    )(a, b)


@functools.partial(
    pl.kernel,
    out_type=jax.ShapeDtypeStruct((NC * N_NODES, D), jnp.float32),
    mesh=plsc.VectorSubcoreMesh(core_axis_name="c", subcore_axis_name="s"),
    scratch_types=[
        pltpu.VMEM((2, CHUNK), jnp.int32),
        pltpu.VMEM((2, CHUNK), jnp.int32),
        pltpu.VMEM((CHUNK,), jnp.float32),
        pltpu.VMEM((CHUNK,), jnp.float32),
        pltpu.VMEM((CHUNK, D), jnp.float32),
        pltpu.VMEM((CHUNK, D), jnp.float32),
        pltpu.VMEM_SHARED((N_NODES, D), jnp.float32),
        pltpu.SemaphoreType.DMA,
        pltpu.SemaphoreType.DMA,
        pltpu.SemaphoreType.DMA,
        pltpu.SemaphoreType.DMA,
    ],
)
def _sc_spmm(ed_hbm, ev_hbm, support_hbm, out_hbm,
             ed0, ed1, ev0, ev1, rows0, rows1, acc_sh,
             sem_e0, sem_e1, sem_g0, sem_g1):
    c = lax.axis_index("c")
    s = lax.axis_index("s")
    wid = c * NS + s

    def zero_row(e, _):
        for j in range(D_GROUPS):
            rows0[e, pl.ds(j * LANES, LANES)] = jnp.zeros((LANES,),
                                                          jnp.float32)
        return _

    lax.fori_loop(0, CHUNK, zero_row, None)
    base = s * ROWS_MAIN
    rem = ROWS_MAIN % CHUNK
    for t in range(ROWS_MAIN // CHUNK):
        pltpu.sync_copy(rows0, acc_sh.at[pl.ds(base + t * CHUNK, CHUNK)])
    pltpu.sync_copy(rows0.at[pl.ds(0, rem)],
                    acc_sh.at[pl.ds(base + ROWS_MAIN - rem, rem)])

    @pl.when(s == 0)
    def _zero_tail():
        pltpu.sync_copy(rows0.at[pl.ds(0, ROWS_TAIL)],
                        acc_sh.at[pl.ds(NS * ROWS_MAIN, ROWS_TAIL)])

    plsc.subcore_barrier()

    def scale(rows, ev):
        def g_body(g, _):
            evg = ev[pl.ds(g * LANES, LANES)]
            e0 = g * LANES
            for l in range(LANES):
                val = evg[l]
                for j in range(D_GROUPS):
                    sl = pl.ds(j * LANES, LANES)
                    rows[e0 + l, sl] = rows[e0 + l, sl] * val
            return _

        lax.fori_loop(0, CHUNK // LANES, g_body, None)

    pltpu.sync_copy(ev_hbm.at[wid, 0], ev0)
    pltpu.async_copy(ed_hbm.at[wid, 0], ed0, sem_e0).wait()
    pltpu.async_copy(support_hbm.at[ed0.at[0]], rows0, sem_g0)
    pltpu.sync_copy(ev_hbm.at[wid, 1], ev1)
    pltpu.async_copy(ed_hbm.at[wid, 1], ed1, sem_e1)

    def pair_body(p, _):
        i = 2 * p
        pltpu.make_async_copy(ed_hbm.at[wid, i + 1], ed1, sem_e1).wait()
        pltpu.async_copy(support_hbm.at[ed1.at[0]], rows1, sem_g1)

        pltpu.make_async_copy(support_hbm.at[ed0.at[0]], rows0, sem_g0).wait()
        scale(rows0, ev0)
        pltpu.sync_copy(rows0, acc_sh.at[ed0.at[1]], add=True)

        @pl.when(p < NPAIRS - 1)
        def _preload_even():
            pltpu.sync_copy(ev_hbm.at[wid, i + 2], ev0)
            pltpu.async_copy(ed_hbm.at[wid, i + 2], ed0, sem_e0).wait()
            pltpu.async_copy(support_hbm.at[ed0.at[0]], rows0, sem_g0)

        pltpu.make_async_copy(support_hbm.at[ed1.at[0]], rows1, sem_g1).wait()
        scale(rows1, ev1)
        pltpu.sync_copy(rows1, acc_sh.at[ed1.at[1]], add=True)

        @pl.when(p < NPAIRS - 1)
        def _preload_odd():
            pltpu.sync_copy(ev_hbm.at[wid, i + 3], ev1)
            pltpu.async_copy(ed_hbm.at[wid, i + 3], ed1, sem_e1)

        return _

    lax.fori_loop(0, NPAIRS, pair_body, None)
    plsc.subcore_barrier()

    out_base = c * N_NODES + base
    for t in range(ROWS_MAIN // CHUNK):
        pltpu.sync_copy(acc_sh.at[pl.ds(base + t * CHUNK, CHUNK)], rows0)
        pltpu.sync_copy(rows0, out_hbm.at[pl.ds(out_base + t * CHUNK, CHUNK)])
    pltpu.sync_copy(acc_sh.at[pl.ds(base + ROWS_MAIN - rem, rem)],
                    rows0.at[pl.ds(0, rem)])
    pltpu.sync_copy(rows0.at[pl.ds(0, rem)],
                    out_hbm.at[pl.ds(out_base + ROWS_MAIN - rem, rem)])

    @pl.when(s == 0)
    def _drain_tail():
        pltpu.sync_copy(acc_sh.at[pl.ds(NS * ROWS_MAIN, ROWS_TAIL)],
                        rows1.at[pl.ds(0, ROWS_TAIL)])
        pltpu.sync_copy(rows1.at[pl.ds(0, ROWS_TAIL)],
                        out_hbm.at[pl.ds(c * N_NODES + NS * ROWS_MAIN,
                                         ROWS_TAIL)])


def kernel(edge_index, edge_values, input_feature, weight):
    zi = jnp.arange(E_PAD, dtype=jnp.int32) % N_NODES
    row = jnp.concatenate([edge_index[0].astype(jnp.int32), zi])
    col = jnp.concatenate([edge_index[1].astype(jnp.int32), zi])
    ev = jnp.concatenate([edge_values, jnp.zeros((E_PAD,), jnp.float32)])
    shp = (NW, N_CHUNKS, CHUNK)
    ed = jnp.stack([col.reshape(shp), row.reshape(shp)], axis=2)
    support = _matmul_call(input_feature, weight)
    partials = _sc_spmm(ed, ev.reshape(shp), support)
    return _add_call(partials[:N_NODES], partials[N_NODES:])


# speedup vs baseline: 2.5481x; 1.1663x over previous
import functools

import jax
import jax.numpy as jnp
from jax import lax
from jax.experimental import pallas as pl
from jax.experimental.pallas import tpu as pltpu
from jax.experimental.pallas import tpu_sc as plsc

N_NODES = 10000
N_EDGES = 320000
D = 128

NC = 2
NS = 16
NW = NC * NS
LANES = 16

E_PER_W = N_EDGES // NW
CHUNK = 128
N_CHUNKS = E_PER_W // CHUNK
NPAIRS = N_CHUNKS // 2
TAIL = E_PER_W - N_CHUNKS * CHUNK
ROWS_MAIN = 624
ROWS_TAIL = N_NODES - NS * ROWS_MAIN
D_GROUPS = D // LANES


def _matmul_call(x, w):
    def mm_body(x_ref, w_ref, o_ref):
        o_ref[...] = jnp.dot(x_ref[...], w_ref[...],
                             preferred_element_type=jnp.float32)

    return pl.pallas_call(
        mm_body,
        grid=(10,),
        in_specs=[
            pl.BlockSpec((N_NODES // 10, D), lambda i: (i, 0)),
            pl.BlockSpec((D, D), lambda i: (0, 0)),
        ],
        out_specs=pl.BlockSpec((N_NODES // 10, D), lambda i: (i, 0)),
        out_shape=jax.ShapeDtypeStruct((N_NODES, D), jnp.float32),
    )(x, w)


def _add_call(a, b):
    def add_body(a_ref, b_ref, o_ref):
        o_ref[...] = a_ref[...] + b_ref[...]

    return pl.pallas_call(
        add_body,
        grid=(10,),
        in_specs=[
            pl.BlockSpec((N_NODES // 10, D), lambda i: (i, 0)),
            pl.BlockSpec((N_NODES // 10, D), lambda i: (i, 0)),
        ],
        out_specs=pl.BlockSpec((N_NODES // 10, D), lambda i: (i, 0)),
        out_shape=jax.ShapeDtypeStruct((N_NODES, D), jnp.float32),
    )(a, b)


@functools.partial(
    pl.kernel,
    out_type=jax.ShapeDtypeStruct((NC * N_NODES, D), jnp.float32),
    mesh=plsc.VectorSubcoreMesh(core_axis_name="c", subcore_axis_name="s"),
    scratch_types=[
        pltpu.VMEM((CHUNK,), jnp.int32),
        pltpu.VMEM((CHUNK,), jnp.int32),
        pltpu.VMEM((CHUNK,), jnp.int32),
        pltpu.VMEM((CHUNK,), jnp.int32),
        pltpu.VMEM((CHUNK,), jnp.float32),
        pltpu.VMEM((CHUNK,), jnp.float32),
        pltpu.VMEM((CHUNK, D), jnp.float32),
        pltpu.VMEM((CHUNK, D), jnp.float32),
        pltpu.VMEM((TAIL,), jnp.int32),
        pltpu.VMEM((TAIL,), jnp.int32),
        pltpu.VMEM((TAIL,), jnp.float32),
        pltpu.VMEM_SHARED((N_NODES, D), jnp.float32),
        pltpu.SemaphoreType.DMA,
        pltpu.SemaphoreType.DMA,
        pltpu.SemaphoreType.DMA,
        pltpu.SemaphoreType.DMA,
        pltpu.SemaphoreType.DMA,
        pltpu.SemaphoreType.DMA,
    ],
)
def _sc_spmm(col_hbm, row_hbm, ev_hbm, support_hbm, out_hbm,
             col0, col1, row0, row1, ev0, ev1, rows0, rows1,
             colt, rowt, evt, acc_sh,
             sem_e0, sem_e1, sem_g0, sem_g1, sem_s0, sem_s1):
    c = lax.axis_index("c")
    s = lax.axis_index("s")
    wid = c * NS + s

    def zero_row(e, _):
        for j in range(D_GROUPS):
            rows0[e, pl.ds(j * LANES, LANES)] = jnp.zeros((LANES,),
                                                          jnp.float32)
        return _

    lax.fori_loop(0, CHUNK, zero_row, None)
    base = s * ROWS_MAIN
    rem = ROWS_MAIN % CHUNK
    for t in range(ROWS_MAIN // CHUNK):
        pltpu.sync_copy(rows0, acc_sh.at[pl.ds(base + t * CHUNK, CHUNK)])
    pltpu.sync_copy(rows0.at[pl.ds(0, rem)],
                    acc_sh.at[pl.ds(base + ROWS_MAIN - rem, rem)])

    @pl.when(s == 0)
    def _zero_tail():
        pltpu.sync_copy(rows0.at[pl.ds(0, ROWS_TAIL)],
                        acc_sh.at[pl.ds(NS * ROWS_MAIN, ROWS_TAIL)])

    plsc.subcore_barrier()

    def scale(rows, ev):
        def g_body(g, _):
            evg = ev[pl.ds(g * LANES, LANES)]
            e0 = g * LANES
            for l in range(LANES):
                val = evg[l]
                for j in range(D_GROUPS):
                    sl = pl.ds(j * LANES, LANES)
                    rows[e0 + l, sl] = rows[e0 + l, sl] * val
            return _

        lax.fori_loop(0, CHUNK // LANES, g_body, None)

    def load_edges(i, colb, rowb, evb, sem):
        sl = pl.ds(i * CHUNK, CHUNK)
        pltpu.async_copy(col_hbm.at[wid, sl], colb, sem)
        pltpu.async_copy(row_hbm.at[wid, sl], rowb, sem)
        pltpu.async_copy(ev_hbm.at[wid, sl], evb, sem)
        pltpu.make_async_copy(col_hbm.at[wid, sl], colb, sem).wait()
        pltpu.make_async_copy(row_hbm.at[wid, sl], rowb, sem).wait()
        pltpu.make_async_copy(ev_hbm.at[wid, sl], evb, sem).wait()

    load_edges(0, col0, row0, ev0, sem_e0)
    pltpu.async_copy(support_hbm.at[col0], rows0, sem_g0)
    load_edges(1, col1, row1, ev1, sem_e1)
    pltpu.async_copy(support_hbm.at[col1], rows1, sem_g1)

    def pair_body(p, _):
        i = 2 * p
        pltpu.make_async_copy(support_hbm.at[col0], rows0, sem_g0).wait()
        scale(rows0, ev0)
        pltpu.async_copy(rows0, acc_sh.at[row0], sem_s0, add=True)

        pltpu.make_async_copy(support_hbm.at[col1], rows1, sem_g1).wait()
        scale(rows1, ev1)
        pltpu.async_copy(rows1, acc_sh.at[row1], sem_s1, add=True)

        pltpu.make_async_copy(rows0, acc_sh.at[row0], sem_s0).wait()

        @pl.when(p < NPAIRS - 1)
        def _preload_even():
            load_edges(i + 2, col0, row0, ev0, sem_e0)
            pltpu.async_copy(support_hbm.at[col0], rows0, sem_g0)

        pltpu.make_async_copy(rows1, acc_sh.at[row1], sem_s1).wait()

        @pl.when(p < NPAIRS - 1)
        def _preload_odd():
            load_edges(i + 3, col1, row1, ev1, sem_e1)
            pltpu.async_copy(support_hbm.at[col1], rows1, sem_g1)

        return _

    lax.fori_loop(0, NPAIRS, pair_body, None)

    tl = pl.ds(N_CHUNKS * CHUNK, TAIL)
    pltpu.sync_copy(col_hbm.at[wid, tl], colt)
    pltpu.sync_copy(row_hbm.at[wid, tl], rowt)
    pltpu.sync_copy(ev_hbm.at[wid, tl], evt)
    pltpu.async_copy(support_hbm.at[colt], rows0.at[pl.ds(0, TAIL)],
                     sem_g0).wait()
    evg = evt[...]
    for l in range(TAIL):
        val = evg[l]
        for j in range(D_GROUPS):
            sl = pl.ds(j * LANES, LANES)
            rows0[l, sl] = rows0[l, sl] * val
    pltpu.sync_copy(rows0.at[pl.ds(0, TAIL)], acc_sh.at[rowt], add=True)

    plsc.subcore_barrier()

    out_base = c * N_NODES + base
    for t in range(ROWS_MAIN // CHUNK):
        pltpu.sync_copy(acc_sh.at[pl.ds(base + t * CHUNK, CHUNK)], rows0)
        pltpu.sync_copy(rows0, out_hbm.at[pl.ds(out_base + t * CHUNK, CHUNK)])
    pltpu.sync_copy(acc_sh.at[pl.ds(base + ROWS_MAIN - rem, rem)],
                    rows0.at[pl.ds(0, rem)])
    pltpu.sync_copy(rows0.at[pl.ds(0, rem)],
                    out_hbm.at[pl.ds(out_base + ROWS_MAIN - rem, rem)])

    @pl.when(s == 0)
    def _drain_tail():
        pltpu.sync_copy(acc_sh.at[pl.ds(NS * ROWS_MAIN, ROWS_TAIL)],
                        rows1.at[pl.ds(0, ROWS_TAIL)])
        pltpu.sync_copy(rows1.at[pl.ds(0, ROWS_TAIL)],
                        out_hbm.at[pl.ds(c * N_NODES + NS * ROWS_MAIN,
                                         ROWS_TAIL)])


def kernel(edge_index, edge_values, input_feature, weight):
    ei = edge_index.astype(jnp.int32)
    row = ei[0].reshape(NW, E_PER_W)
    col = ei[1].reshape(NW, E_PER_W)
    ev = edge_values.reshape(NW, E_PER_W)
    support = _matmul_call(input_feature, weight)
    partials = _sc_spmm(col, row, ev, support)
    return _add_call(partials[:N_NODES], partials[N_NODES:])


# speedup vs baseline: 2.6052x; 1.0224x over previous
import functools

import jax
import jax.numpy as jnp
from jax import lax
from jax.experimental import pallas as pl
from jax.experimental.pallas import tpu as pltpu
from jax.experimental.pallas import tpu_sc as plsc

N_NODES = 10000
N_EDGES = 320000
D = 128

NC = 2
NS = 16
NW = NC * NS
LANES = 16

E_PER_W = N_EDGES // NW
CHUNK = 128
N_CHUNKS = E_PER_W // CHUNK
NPAIRS = N_CHUNKS // 2
TAIL = E_PER_W - N_CHUNKS * CHUNK
ROWS_MAIN = 624
ROWS_TAIL = N_NODES - NS * ROWS_MAIN
D_GROUPS = D // LANES


def _matmul_call(x, w):
    def mm_body(x_ref, w_ref, o_ref):
        o_ref[...] = jnp.dot(x_ref[...], w_ref[...],
                             preferred_element_type=jnp.float32)

    return pl.pallas_call(
        mm_body,
        grid=(5,),
        in_specs=[
            pl.BlockSpec((N_NODES // 5, D), lambda i: (i, 0)),
            pl.BlockSpec((D, D), lambda i: (0, 0)),
        ],
        out_specs=pl.BlockSpec((N_NODES // 5, D), lambda i: (i, 0)),
        out_shape=jax.ShapeDtypeStruct((N_NODES, D), jnp.float32),
    )(x, w)


def _add_call(a, b):
    def add_body(a_ref, b_ref, o_ref):
        o_ref[...] = a_ref[...] + b_ref[...]

    return pl.pallas_call(
        add_body,
        grid=(5,),
        in_specs=[
            pl.BlockSpec((N_NODES // 5, D), lambda i: (i, 0)),
            pl.BlockSpec((N_NODES // 5, D), lambda i: (i, 0)),
        ],
        out_specs=pl.BlockSpec((N_NODES // 5, D), lambda i: (i, 0)),
        out_shape=jax.ShapeDtypeStruct((N_NODES, D), jnp.float32),
    )(a, b)


@functools.partial(
    pl.kernel,
    out_type=jax.ShapeDtypeStruct((NC * N_NODES, D), jnp.float32),
    mesh=plsc.VectorSubcoreMesh(core_axis_name="c", subcore_axis_name="s"),
    scratch_types=[
        pltpu.VMEM((CHUNK,), jnp.int32),
        pltpu.VMEM((CHUNK,), jnp.int32),
        pltpu.VMEM((CHUNK,), jnp.int32),
        pltpu.VMEM((CHUNK,), jnp.int32),
        pltpu.VMEM((CHUNK,), jnp.float32),
        pltpu.VMEM((CHUNK,), jnp.float32),
        pltpu.VMEM((CHUNK, D), jnp.float32),
        pltpu.VMEM((CHUNK, D), jnp.float32),
        pltpu.VMEM((TAIL,), jnp.int32),
        pltpu.VMEM((TAIL,), jnp.int32),
        pltpu.VMEM((TAIL,), jnp.float32),
        pltpu.VMEM_SHARED((N_NODES, D), jnp.float32),
        pltpu.SemaphoreType.DMA,
        pltpu.SemaphoreType.DMA,
        pltpu.SemaphoreType.DMA,
        pltpu.SemaphoreType.DMA,
        pltpu.SemaphoreType.DMA,
        pltpu.SemaphoreType.DMA,
    ],
)
def _sc_spmm(col_hbm, row_hbm, ev_hbm, support_hbm, out_hbm,
             col0, col1, row0, row1, ev0, ev1, rows0, rows1,
             colt, rowt, evt, acc_sh,
             sem_e0, sem_e1, sem_g0, sem_g1, sem_s0, sem_s1):
    c = lax.axis_index("c")
    s = lax.axis_index("s")
    wid = c * NS + s

    def zero_row(e, _):
        for j in range(D_GROUPS):
            rows0[e, pl.ds(j * LANES, LANES)] = jnp.zeros((LANES,),
                                                          jnp.float32)
        return _

    lax.fori_loop(0, CHUNK, zero_row, None)
    base = s * ROWS_MAIN
    rem = ROWS_MAIN % CHUNK
    for t in range(ROWS_MAIN // CHUNK):
        pltpu.sync_copy(rows0, acc_sh.at[pl.ds(base + t * CHUNK, CHUNK)])
    pltpu.sync_copy(rows0.at[pl.ds(0, rem)],
                    acc_sh.at[pl.ds(base + ROWS_MAIN - rem, rem)])

    @pl.when(s == 0)
    def _zero_tail():
        pltpu.sync_copy(rows0.at[pl.ds(0, ROWS_TAIL)],
                        acc_sh.at[pl.ds(NS * ROWS_MAIN, ROWS_TAIL)])

    plsc.subcore_barrier()

    def scale(rows, ev):
        def g_body(g, _):
            evg = ev[pl.ds(g * LANES, LANES)]
            e0 = g * LANES
            for l in range(LANES):
                val = evg[l]
                for j in range(D_GROUPS):
                    sl = pl.ds(j * LANES, LANES)
                    rows[e0 + l, sl] = rows[e0 + l, sl] * val
            return _

        lax.fori_loop(0, CHUNK // LANES, g_body, None)

    def load_edges(i, colb, rowb, evb, sem):
        sl = pl.ds(i * CHUNK, CHUNK)
        pltpu.async_copy(col_hbm.at[wid, sl], colb, sem)
        pltpu.async_copy(row_hbm.at[wid, sl], rowb, sem)
        pltpu.async_copy(ev_hbm.at[wid, sl], evb, sem)
        pltpu.make_async_copy(col_hbm.at[wid, sl], colb, sem).wait()
        pltpu.make_async_copy(row_hbm.at[wid, sl], rowb, sem).wait()
        pltpu.make_async_copy(ev_hbm.at[wid, sl], evb, sem).wait()

    load_edges(0, col0, row0, ev0, sem_e0)
    pltpu.async_copy(support_hbm.at[col0], rows0, sem_g0)
    load_edges(1, col1, row1, ev1, sem_e1)
    pltpu.async_copy(support_hbm.at[col1], rows1, sem_g1)

    def pair_body(p, _):
        i = 2 * p
        pltpu.make_async_copy(support_hbm.at[col0], rows0, sem_g0).wait()
        scale(rows0, ev0)
        pltpu.async_copy(rows0, acc_sh.at[row0], sem_s0, add=True)

        pltpu.make_async_copy(support_hbm.at[col1], rows1, sem_g1).wait()
        scale(rows1, ev1)
        pltpu.async_copy(rows1, acc_sh.at[row1], sem_s1, add=True)

        pltpu.make_async_copy(rows0, acc_sh.at[row0], sem_s0).wait()

        @pl.when(p < NPAIRS - 1)
        def _preload_even():
            load_edges(i + 2, col0, row0, ev0, sem_e0)
            pltpu.async_copy(support_hbm.at[col0], rows0, sem_g0)

        pltpu.make_async_copy(rows1, acc_sh.at[row1], sem_s1).wait()

        @pl.when(p < NPAIRS - 1)
        def _preload_odd():
            load_edges(i + 3, col1, row1, ev1, sem_e1)
            pltpu.async_copy(support_hbm.at[col1], rows1, sem_g1)

        return _

    lax.fori_loop(0, NPAIRS, pair_body, None)

    tl = pl.ds(N_CHUNKS * CHUNK, TAIL)
    pltpu.sync_copy(col_hbm.at[wid, tl], colt)
    pltpu.sync_copy(row_hbm.at[wid, tl], rowt)
    pltpu.sync_copy(ev_hbm.at[wid, tl], evt)
    pltpu.async_copy(support_hbm.at[colt], rows0.at[pl.ds(0, TAIL)],
                     sem_g0).wait()
    evg = evt[...]
    for l in range(TAIL):
        val = evg[l]
        for j in range(D_GROUPS):
            sl = pl.ds(j * LANES, LANES)
            rows0[l, sl] = rows0[l, sl] * val
    pltpu.sync_copy(rows0.at[pl.ds(0, TAIL)], acc_sh.at[rowt], add=True)

    plsc.subcore_barrier()

    out_base = c * N_NODES + base
    pltpu.sync_copy(acc_sh.at[pl.ds(base, ROWS_MAIN)],
                    out_hbm.at[pl.ds(out_base, ROWS_MAIN)])

    @pl.when(s == 0)
    def _drain_tail():
        pltpu.sync_copy(acc_sh.at[pl.ds(NS * ROWS_MAIN, ROWS_TAIL)],
                        out_hbm.at[pl.ds(c * N_NODES + NS * ROWS_MAIN,
                                         ROWS_TAIL)])


def kernel(edge_index, edge_values, input_feature, weight):
    ei = edge_index.astype(jnp.int32)
    row = ei[0].reshape(NW, E_PER_W)
    col = ei[1].reshape(NW, E_PER_W)
    ev = edge_values.reshape(NW, E_PER_W)
    support = _matmul_call(input_feature, weight)
    partials = _sc_spmm(col, row, ev, support)
    return _add_call(partials[:N_NODES], partials[N_NODES:])
